# baseline (speedup 1.0000x reference)
import jax
import jax.numpy as jnp
from jax import lax
from jax.experimental import pallas as pl
from jax.experimental.pallas import tpu as pltpu
from jax.experimental.pallas import tpu_sc as plsc

N_PTS = 524288
C_DIM = 96
N_CELLS = 262144
NG = 8
G_SHIFT = 15
GB = 4
NB = NG * GB
BUCKET = N_CELLS // NB
L_SHIFT = 13
NC = 2
NS = 16
P = N_PTS // NS
GCAP = P + 16
FB = 128
MCAP = 2 * FB
TS = BUCKET // NS
DC = 128
SB = 2048


def _body(feat_hbm, gi_hbm, sums_hbm, cnts_hbm,
          gbuf, glist, mptr, lcnt, idbuf, gidxa, gidxb, lidx, rowbuf,
          dbuf, zc, acc, scnt, sem_g, sem_s):
    c = lax.axis_index("c")
    s = lax.axis_index("s")
    tbase = s * P
    iota16 = lax.iota(jnp.int32, 16)
    zeros16 = jnp.zeros((16,), jnp.float32)
    ones16 = jnp.ones((16,), jnp.float32)
    neg16 = jnp.full((16,), -1, jnp.int32)

    def zcrow(r, _):
        zc[r] = zeros16
        return 0
    lax.fori_loop(0, 32, zcrow, 0)

    def idrow(q, _):
        for i in range(8):
            idbuf[q, pl.ds(i * 16, 16)] = q * 128 + i * 16 + iota16
        return 0
    lax.fori_loop(0, (BUCKET // 16) // 128, idrow, 0)

    def drain_scatter():
        pltpu.make_async_copy(rowbuf, acc.at[lidx], sem_s).wait()

    def flush_once(fcnt):
        @pl.when(fcnt > 0)
        def _():
            drain_scatter()
        for q in range(FB // 16):
            r16 = mptr[pl.ds(q * 16, 16)]
            valid = r16 >= 0
            loc = jnp.where(valid,
                            jnp.right_shift(r16, 15),
                            jnp.int32(BUCKET))
            rel = jnp.bitwise_and(r16, 32767)
            gi_half = gidxa if q < (FB // 32) else gidxb
            gi_half[pl.ds((q % (FB // 32)) * 16, 16)] = tbase + rel
            lidx[pl.ds(q * 16, 16)] = loc
        cpa = pltpu.async_copy(
            feat_hbm.at[gidxa], rowbuf.at[pl.ds(0, FB // 2)], sem_g)
        cpb = pltpu.async_copy(
            feat_hbm.at[gidxb], rowbuf.at[pl.ds(FB // 2, FB // 2)], sem_g)
        cpa.wait()
        cpb.wait()
        pltpu.async_copy(rowbuf, acc.at[lidx], sem_s, add=True)

    def group_body(gi, _):
        g = c * (NG // NC) + gi

        def stream_body(ch, gcnt):
            pltpu.sync_copy(gi_hbm.at[pl.ds(tbase + ch * SB, SB)], gbuf)

            def scan_body(i, cnt):
                v = gbuf[pl.ds(i * 16, 16)]
                m = jnp.right_shift(v, G_SHIFT) == g
                rel = ch * SB + i * 16 + iota16
                e = jnp.bitwise_or(
                    jnp.left_shift(jnp.bitwise_and(v, 32767), 15), rel)
                pos = cnt + plsc.cumsum(m.astype(jnp.int32)) - 1
                plsc.store_scatter(glist, [pos], e, mask=m)
                return cnt + jnp.sum(m.astype(jnp.int32))
            return lax.fori_loop(0, SB // 16, scan_body, gcnt)
        gcnt = lax.fori_loop(0, P // SB, stream_body, jnp.int32(0))
        plsc.store_scatter(glist, [gcnt + iota16], neg16)
        gsteps = jnp.right_shift(gcnt + 15, 4)

        def bucket_body(sub, _):
            b = g * GB + sub

            def zl(i, _):
                lcnt[i] = zeros16
                return 0
            lax.fori_loop(0, BUCKET // 16, zl, 0)

            def zrow(r, _):
                for q in range(C_DIM // 16):
                    dbuf[r, pl.ds(q * 16, 16)] = zeros16
                return 0
            lax.fori_loop(0, DC, zrow, 0)
            for kk in range(TS // DC):
                pltpu.sync_copy(dbuf, acc.at[pl.ds(s * TS + kk * DC, DC)])
            pltpu.sync_copy(zc, scnt.at[pl.ds(s * 32, 32)])
            plsc.subcore_barrier()

            def bscan(i, carry):
                cnt, fcnt = carry
                e = glist[pl.ds(i * 16, 16)]
                cell15 = jnp.right_shift(e, 15)
                m = jnp.logical_and(
                    jnp.right_shift(cell15, L_SHIFT) == sub, e >= 0)
                loc = jnp.bitwise_and(cell15, BUCKET - 1)
                plsc.addupdate_scatter(
                    lcnt, [jnp.right_shift(loc, 4),
                           jnp.bitwise_and(loc, 15)], ones16, mask=m)
                packed = jnp.bitwise_or(
                    jnp.left_shift(loc, 15), jnp.bitwise_and(e, 32767))
                pos = cnt + plsc.cumsum(m.astype(jnp.int32)) - 1
                plsc.store_scatter(mptr, [pos], packed, mask=m)
                cnt2 = cnt + jnp.sum(m.astype(jnp.int32))
                full = cnt2 >= FB

                @pl.when(full)
                def _():
                    flush_once(fcnt)
                    ov = mptr[pl.ds(FB, 16)]
                    mptr[pl.ds(0, 16)] = ov
                cnt3 = jnp.where(full, cnt2 - FB, cnt2)
                return cnt3, fcnt + full.astype(jnp.int32)
            cnt, fcnt = lax.fori_loop(0, gsteps, bscan,
                                      (jnp.int32(0), jnp.int32(0)))

            def pad_body(q, _):
                plsc.store_scatter(mptr, [cnt + q * 16 + iota16], neg16)
                return 0
            lax.fori_loop(0, 8, pad_body, 0)

            @pl.when(cnt > 0)
            def _():
                flush_once(fcnt)
            ffin = fcnt + (cnt > 0).astype(jnp.int32)

            @pl.when(ffin > 0)
            def _():
                drain_scatter()

            for q in range((BUCKET // 16) // 128):
                pltpu.sync_copy(lcnt.at[pl.ds(q * 128, 128)],
                                scnt.at[idbuf.at[q]], add=True)
            plsc.subcore_barrier()

            def dchunk(kk, _):
                row0 = s * TS + kk * DC
                pltpu.sync_copy(acc.at[pl.ds(row0, DC)],
                                sums_hbm.at[pl.ds(b * BUCKET + row0, DC)])
                return 0
            lax.fori_loop(0, TS // DC, dchunk, 0)
            pltpu.sync_copy(scnt.at[pl.ds(s * 32, 32)],
                            cnts_hbm.at[pl.ds(b * (BUCKET // 16) + s * 32,
                                              32)])

            plsc.subcore_barrier()
            return 0
        lax.fori_loop(0, GB, bucket_body, 0)
        return 0
    lax.fori_loop(0, NG // NC, group_body, 0)


def _div_body(s_ref, c_ref, o_ref):
    o_ref[...] = s_ref[...] / jnp.maximum(c_ref[...], 1.0)


@jax.jit
def kernel(feat, grid_index):
    run = pl.kernel(
        _body,
        out_type=(
            jax.ShapeDtypeStruct((N_CELLS, C_DIM), jnp.float32),
            jax.ShapeDtypeStruct((N_CELLS // 16, 16), jnp.float32),
        ),
        mesh=plsc.VectorSubcoreMesh(core_axis_name="c", subcore_axis_name="s"),
        compiler_params=pltpu.CompilerParams(
            needs_layout_passes=False, use_tc_tiling_on_sc=False),
        scratch_types=[
            pltpu.VMEM((SB,), jnp.int32),
            pltpu.VMEM((GCAP,), jnp.int32),
            pltpu.VMEM((MCAP,), jnp.int32),
            pltpu.VMEM((BUCKET // 16, 16), jnp.float32),
            pltpu.VMEM(((BUCKET // 16) // 128, 128), jnp.int32),
            pltpu.VMEM((FB // 2,), jnp.int32),
            pltpu.VMEM((FB // 2,), jnp.int32),
            pltpu.VMEM((FB,), jnp.int32),
            pltpu.VMEM((FB, C_DIM), jnp.float32),
            pltpu.VMEM((DC, C_DIM), jnp.float32),
            pltpu.VMEM((32, 16), jnp.float32),
            pltpu.VMEM_SHARED((BUCKET + 8, C_DIM), jnp.float32),
            pltpu.VMEM_SHARED((BUCKET // 16, 16), jnp.float32),
            pltpu.SemaphoreType.DMA,
            pltpu.SemaphoreType.DMA,
        ],
    )
    sums, cnts = run(feat, grid_index)
    cnts_col = cnts.reshape(N_CELLS, 1)
    RB = 2048
    return pl.pallas_call(
        _div_body,
        out_shape=jax.ShapeDtypeStruct((N_CELLS, C_DIM), jnp.float32),
        grid=(N_CELLS // RB,),
        in_specs=[
            pl.BlockSpec((RB, C_DIM), lambda i: (i, 0)),
            pl.BlockSpec((RB, 1), lambda i: (i, 0)),
        ],
        out_specs=pl.BlockSpec((RB, C_DIM), lambda i: (i, 0)),
    )(sums, cnts_col)


# speedup vs baseline: 1.0178x; 1.0178x over previous
import jax
import jax.numpy as jnp
from jax import lax
from jax.experimental import pallas as pl
from jax.experimental.pallas import tpu as pltpu
from jax.experimental.pallas import tpu_sc as plsc

N_PTS = 524288
C_DIM = 96
N_CELLS = 262144
NG = 8
G_SHIFT = 15
GB = 4
NB = NG * GB
BUCKET = N_CELLS // NB
L_SHIFT = 13
NC = 2
NS = 16
P = N_PTS // NS
GCAP = P + 16
FB = 128
MCAP = 2 * FB
TS = BUCKET // NS
DC = 128
SB = 2048


def _body(feat_hbm, gi_hbm, sums_hbm, cnts_hbm,
          gbuf, glist, mptr, lcnt, idbuf, gidxa, gidxb, lidx, rowbuf,
          dbuf, zc, zlbuf, acc, scnt, sem_g, sem_s):
    c = lax.axis_index("c")
    s = lax.axis_index("s")
    tbase = s * P
    iota16 = lax.iota(jnp.int32, 16)
    zeros16 = jnp.zeros((16,), jnp.float32)
    ones16 = jnp.ones((16,), jnp.float32)
    neg16 = jnp.full((16,), -1, jnp.int32)

    def zdrow(r, _):
        for q in range(C_DIM // 16):
            dbuf[r, pl.ds(q * 16, 16)] = zeros16
        return 0
    lax.fori_loop(0, DC, zdrow, 0)

    def zcrow(r, _):
        zc[r] = zeros16
        return 0
    lax.fori_loop(0, 32, zcrow, 0)
    pltpu.sync_copy(zc, zlbuf.at[pl.ds(s * 32, 32)])
    plsc.subcore_barrier()

    def idrow(q, _):
        for i in range(8):
            idbuf[q, pl.ds(i * 16, 16)] = q * 128 + i * 16 + iota16
        return 0
    lax.fori_loop(0, (BUCKET // 16) // 128, idrow, 0)

    def drain_scatter():
        pltpu.make_async_copy(rowbuf, acc.at[lidx], sem_s).wait()

    def flush_once(fcnt):
        @pl.when(fcnt > 0)
        def _():
            drain_scatter()
        for q in range(FB // 16):
            r16 = mptr[pl.ds(q * 16, 16)]
            valid = r16 >= 0
            loc = jnp.where(valid,
                            jnp.right_shift(r16, 15),
                            jnp.int32(BUCKET))
            rel = jnp.bitwise_and(r16, 32767)
            gi_half = gidxa if q < (FB // 32) else gidxb
            gi_half[pl.ds((q % (FB // 32)) * 16, 16)] = tbase + rel
            lidx[pl.ds(q * 16, 16)] = loc
        cpa = pltpu.async_copy(
            feat_hbm.at[gidxa], rowbuf.at[pl.ds(0, FB // 2)], sem_g)
        cpb = pltpu.async_copy(
            feat_hbm.at[gidxb], rowbuf.at[pl.ds(FB // 2, FB // 2)], sem_g)
        cpa.wait()
        cpb.wait()
        pltpu.async_copy(rowbuf, acc.at[lidx], sem_s, add=True)

    def group_body(gi, _):
        g = c * (NG // NC) + gi

        def stream_body(ch, gcnt):
            pltpu.sync_copy(gi_hbm.at[pl.ds(tbase + ch * SB, SB)], gbuf)

            def scan_body(i, cnt):
                v = gbuf[pl.ds(i * 16, 16)]
                m = jnp.right_shift(v, G_SHIFT) == g
                rel = ch * SB + i * 16 + iota16
                e = jnp.bitwise_or(
                    jnp.left_shift(jnp.bitwise_and(v, 32767), 15), rel)
                pos = cnt + plsc.cumsum(m.astype(jnp.int32)) - 1
                plsc.store_scatter(glist, [pos], e, mask=m)
                return cnt + jnp.sum(m.astype(jnp.int32))
            return lax.fori_loop(0, SB // 16, scan_body, gcnt)
        gcnt = lax.fori_loop(0, P // SB, stream_body, jnp.int32(0))
        plsc.store_scatter(glist, [gcnt + iota16], neg16)
        gsteps = jnp.right_shift(gcnt + 15, 4)

        def bucket_body(sub, _):
            b = g * GB + sub

            pltpu.sync_copy(zlbuf, lcnt)
            for kk in range(TS // DC):
                pltpu.sync_copy(dbuf, acc.at[pl.ds(s * TS + kk * DC, DC)])
            pltpu.sync_copy(lcnt.at[pl.ds(0, 32)],
                            scnt.at[pl.ds(s * 32, 32)])
            plsc.subcore_barrier()

            def bscan(i, carry):
                cnt, fcnt = carry
                e = glist[pl.ds(i * 16, 16)]
                cell15 = jnp.right_shift(e, 15)
                m = jnp.logical_and(
                    jnp.right_shift(cell15, L_SHIFT) == sub, e >= 0)
                loc = jnp.bitwise_and(cell15, BUCKET - 1)
                plsc.addupdate_scatter(
                    lcnt, [jnp.right_shift(loc, 4),
                           jnp.bitwise_and(loc, 15)], ones16, mask=m)
                packed = jnp.bitwise_or(
                    jnp.left_shift(loc, 15), jnp.bitwise_and(e, 32767))
                pos = cnt + plsc.cumsum(m.astype(jnp.int32)) - 1
                plsc.store_scatter(mptr, [pos], packed, mask=m)
                cnt2 = cnt + jnp.sum(m.astype(jnp.int32))
                full = cnt2 >= FB

                @pl.when(full)
                def _():
                    flush_once(fcnt)
                    ov = mptr[pl.ds(FB, 16)]
                    mptr[pl.ds(0, 16)] = ov
                cnt3 = jnp.where(full, cnt2 - FB, cnt2)
                return cnt3, fcnt + full.astype(jnp.int32)
            cnt, fcnt = lax.fori_loop(0, gsteps, bscan,
                                      (jnp.int32(0), jnp.int32(0)))

            def pad_body(q, _):
                plsc.store_scatter(mptr, [cnt + q * 16 + iota16], neg16)
                return 0
            lax.fori_loop(0, 8, pad_body, 0)

            @pl.when(cnt > 0)
            def _():
                flush_once(fcnt)
            ffin = fcnt + (cnt > 0).astype(jnp.int32)

            @pl.when(ffin > 0)
            def _():
                drain_scatter()

            for q in range((BUCKET // 16) // 128):
                pltpu.sync_copy(lcnt.at[pl.ds(q * 128, 128)],
                                scnt.at[idbuf.at[q]], add=True)
            plsc.subcore_barrier()

            def dchunk(kk, _):
                row0 = s * TS + kk * DC
                pltpu.sync_copy(acc.at[pl.ds(row0, DC)],
                                sums_hbm.at[pl.ds(b * BUCKET + row0, DC)])
                return 0
            lax.fori_loop(0, TS // DC, dchunk, 0)
            pltpu.sync_copy(scnt.at[pl.ds(s * 32, 32)],
                            cnts_hbm.at[pl.ds(b * (BUCKET // 16) + s * 32,
                                              32)])

            plsc.subcore_barrier()
            return 0
        lax.fori_loop(0, GB, bucket_body, 0)
        return 0
    lax.fori_loop(0, NG // NC, group_body, 0)


def _div_body(s_ref, c_ref, o_ref):
    o_ref[...] = s_ref[...] / jnp.maximum(c_ref[...], 1.0)


@jax.jit
def kernel(feat, grid_index):
    run = pl.kernel(
        _body,
        out_type=(
            jax.ShapeDtypeStruct((N_CELLS, C_DIM), jnp.float32),
            jax.ShapeDtypeStruct((N_CELLS // 16, 16), jnp.float32),
        ),
        mesh=plsc.VectorSubcoreMesh(core_axis_name="c", subcore_axis_name="s"),
        compiler_params=pltpu.CompilerParams(
            needs_layout_passes=False, use_tc_tiling_on_sc=False),
        scratch_types=[
            pltpu.VMEM((SB,), jnp.int32),
            pltpu.VMEM((GCAP,), jnp.int32),
            pltpu.VMEM((MCAP,), jnp.int32),
            pltpu.VMEM((BUCKET // 16, 16), jnp.float32),
            pltpu.VMEM(((BUCKET // 16) // 128, 128), jnp.int32),
            pltpu.VMEM((FB // 2,), jnp.int32),
            pltpu.VMEM((FB // 2,), jnp.int32),
            pltpu.VMEM((FB,), jnp.int32),
            pltpu.VMEM((FB, C_DIM), jnp.float32),
            pltpu.VMEM((DC, C_DIM), jnp.float32),
            pltpu.VMEM((32, 16), jnp.float32),
            pltpu.VMEM_SHARED((BUCKET // 16, 16), jnp.float32),
            pltpu.VMEM_SHARED((BUCKET + 8, C_DIM), jnp.float32),
            pltpu.VMEM_SHARED((BUCKET // 16, 16), jnp.float32),
            pltpu.SemaphoreType.DMA,
            pltpu.SemaphoreType.DMA,
        ],
    )
    sums, cnts = run(feat, grid_index)
    cnts_col = cnts.reshape(N_CELLS, 1)
    RB = 2048
    return pl.pallas_call(
        _div_body,
        out_shape=jax.ShapeDtypeStruct((N_CELLS, C_DIM), jnp.float32),
        grid=(N_CELLS // RB,),
        in_specs=[
            pl.BlockSpec((RB, C_DIM), lambda i: (i, 0)),
            pl.BlockSpec((RB, 1), lambda i: (i, 0)),
        ],
        out_specs=pl.BlockSpec((RB, C_DIM), lambda i: (i, 0)),
    )(sums, cnts_col)


# speedup vs baseline: 1.0660x; 1.0474x over previous
import jax
import jax.numpy as jnp
from jax import lax
from jax.experimental import pallas as pl
from jax.experimental.pallas import tpu as pltpu
from jax.experimental.pallas import tpu_sc as plsc

N_PTS = 524288
C_DIM = 96
N_CELLS = 262144
NG = 8
G_SHIFT = 15
GB = 4
NB = NG * GB
BUCKET = N_CELLS // NB
L_SHIFT = 13
NC = 2
NS = 16
P = N_PTS // NS
GCAP = P + 16
FB = 128
MCAP = 2 * FB
TS = BUCKET // NS
DC = 64
SB = 2048


def _body(feat_hbm, gi_hbm, sums_hbm, cnts_hbm,
          gbuf, glist, mptr, lcnt, idbuf,
          gidxa0, gidxb0, lidx0, rowbuf0,
          gidxa1, gidxb1, lidx1, rowbuf1,
          dbuf, zc, zlbuf, acc, scnt, sem_g, sem_s):
    c = lax.axis_index("c")
    s = lax.axis_index("s")
    tbase = s * P
    iota16 = lax.iota(jnp.int32, 16)
    zeros16 = jnp.zeros((16,), jnp.float32)
    ones16 = jnp.ones((16,), jnp.float32)
    neg16 = jnp.full((16,), -1, jnp.int32)

    def zdrow(r, _):
        for q in range(C_DIM // 16):
            dbuf[r, pl.ds(q * 16, 16)] = zeros16
        return 0
    lax.fori_loop(0, DC, zdrow, 0)

    def zcrow(r, _):
        zc[r] = zeros16
        return 0
    lax.fori_loop(0, 32, zcrow, 0)
    pltpu.sync_copy(zc, zlbuf.at[pl.ds(s * 32, 32)])
    plsc.subcore_barrier()

    def idrow(q, _):
        for i in range(8):
            idbuf[q, pl.ds(i * 16, 16)] = q * 128 + i * 16 + iota16
        return 0
    lax.fori_loop(0, (BUCKET // 16) // 128, idrow, 0)

    SLOTS = ((gidxa0, gidxb0, lidx0, rowbuf0),
             (gidxa1, gidxb1, lidx1, rowbuf1))

    def wait_scatter(rowb, lidxb):
        pltpu.make_async_copy(rowb, acc.at[lidxb], sem_s).wait()

    def issue_scatter(rowb, lidxb):
        pltpu.async_copy(rowb, acc.at[lidxb], sem_s, add=True)

    def wait_gathers(ga, gb, rowb):
        pltpu.make_async_copy(
            feat_hbm.at[ga], rowb.at[pl.ds(0, FB // 2)], sem_g).wait()
        pltpu.make_async_copy(
            feat_hbm.at[gb], rowb.at[pl.ds(FB // 2, FB // 2)], sem_g).wait()

    def issue_gathers(ga, gb, rowb):
        pltpu.async_copy(feat_hbm.at[ga], rowb.at[pl.ds(0, FB // 2)], sem_g)
        pltpu.async_copy(
            feat_hbm.at[gb], rowb.at[pl.ds(FB // 2, FB // 2)], sem_g)

    def flush_slot(slot, fcnt):
        ga, gb, lidxb, rowb = SLOTS[slot]
        oga, ogb, olidx, orow = SLOTS[1 - slot]

        @pl.when(fcnt >= 2)
        def _():
            wait_scatter(rowb, lidxb)
        for q in range(FB // 16):
            r16 = mptr[pl.ds(q * 16, 16)]
            valid = r16 >= 0
            loc = jnp.where(valid,
                            jnp.right_shift(r16, 15),
                            jnp.int32(BUCKET))
            rel = jnp.bitwise_and(r16, 32767)
            gi_half = ga if q < (FB // 32) else gb
            gi_half[pl.ds((q % (FB // 32)) * 16, 16)] = tbase + rel
            lidxb[pl.ds(q * 16, 16)] = loc

        @pl.when(fcnt >= 1)
        def _():
            wait_gathers(oga, ogb, orow)
            issue_scatter(orow, olidx)
        issue_gathers(ga, gb, rowb)

    def do_flush(fcnt):
        even = jnp.bitwise_and(fcnt, 1) == 0

        @pl.when(even)
        def _():
            flush_slot(0, fcnt)

        @pl.when(jnp.logical_not(even))
        def _():
            flush_slot(1, fcnt)

    def group_body(gi, _):
        g = c * (NG // NC) + gi

        def stream_body(ch, gcnt):
            pltpu.sync_copy(gi_hbm.at[pl.ds(tbase + ch * SB, SB)], gbuf)

            def scan_body(i, cnt):
                v = gbuf[pl.ds(i * 16, 16)]
                m = jnp.right_shift(v, G_SHIFT) == g
                rel = ch * SB + i * 16 + iota16
                e = jnp.bitwise_or(
                    jnp.left_shift(jnp.bitwise_and(v, 32767), 15), rel)
                pos = cnt + plsc.cumsum(m.astype(jnp.int32)) - 1
                plsc.store_scatter(glist, [pos], e, mask=m)
                return cnt + jnp.sum(m.astype(jnp.int32))
            return lax.fori_loop(0, SB // 16, scan_body, gcnt)
        gcnt = lax.fori_loop(0, P // SB, stream_body, jnp.int32(0))
        plsc.store_scatter(glist, [gcnt + iota16], neg16)
        gsteps = jnp.right_shift(gcnt + 15, 4)

        def bucket_body(sub, _):
            b = g * GB + sub

            pltpu.sync_copy(zlbuf, lcnt)
            for kk in range(TS // DC):
                pltpu.sync_copy(dbuf, acc.at[pl.ds(s * TS + kk * DC, DC)])
            pltpu.sync_copy(lcnt.at[pl.ds(0, 32)],
                            scnt.at[pl.ds(s * 32, 32)])
            plsc.subcore_barrier()

            def bscan(i, carry):
                cnt, fcnt = carry
                e = glist[pl.ds(i * 16, 16)]
                cell15 = jnp.right_shift(e, 15)
                m = jnp.logical_and(
                    jnp.right_shift(cell15, L_SHIFT) == sub, e >= 0)
                loc = jnp.bitwise_and(cell15, BUCKET - 1)
                plsc.addupdate_scatter(
                    lcnt, [jnp.right_shift(loc, 4),
                           jnp.bitwise_and(loc, 15)], ones16, mask=m)
                packed = jnp.bitwise_or(
                    jnp.left_shift(loc, 15), jnp.bitwise_and(e, 32767))
                pos = cnt + plsc.cumsum(m.astype(jnp.int32)) - 1
                plsc.store_scatter(mptr, [pos], packed, mask=m)
                cnt2 = cnt + jnp.sum(m.astype(jnp.int32))
                full = cnt2 >= FB

                @pl.when(full)
                def _():
                    do_flush(fcnt)
                    ov = mptr[pl.ds(FB, 16)]
                    mptr[pl.ds(0, 16)] = ov
                cnt3 = jnp.where(full, cnt2 - FB, cnt2)
                return cnt3, fcnt + full.astype(jnp.int32)
            cnt, fcnt = lax.fori_loop(0, gsteps, bscan,
                                      (jnp.int32(0), jnp.int32(0)))

            def pad_body(q, _):
                plsc.store_scatter(mptr, [cnt + q * 16 + iota16], neg16)
                return 0
            lax.fori_loop(0, 8, pad_body, 0)

            @pl.when(cnt > 0)
            def _():
                do_flush(fcnt)
            ftot = fcnt + (cnt > 0).astype(jnp.int32)

            def drain_par(slot, nf):
                ga, gb, lidxb, rowb = SLOTS[slot]
                oga, ogb, olidx, orow = SLOTS[1 - slot]
                wait_gathers(ga, gb, rowb)
                issue_scatter(rowb, lidxb)

                @pl.when(nf >= 2)
                def _():
                    wait_scatter(orow, olidx)
                wait_scatter(rowb, lidxb)

            lastp = jnp.bitwise_and(ftot - 1, 1)

            @pl.when(jnp.logical_and(ftot >= 1, lastp == 0))
            def _():
                drain_par(0, ftot)

            @pl.when(jnp.logical_and(ftot >= 1, lastp == 1))
            def _():
                drain_par(1, ftot)

            for q in range((BUCKET // 16) // 128):
                pltpu.sync_copy(lcnt.at[pl.ds(q * 128, 128)],
                                scnt.at[idbuf.at[q]], add=True)
            plsc.subcore_barrier()

            def dchunk(kk, _):
                row0 = s * TS + kk * DC
                pltpu.sync_copy(acc.at[pl.ds(row0, DC)],
                                sums_hbm.at[pl.ds(b * BUCKET + row0, DC)])
                return 0
            lax.fori_loop(0, TS // DC, dchunk, 0)
            pltpu.sync_copy(scnt.at[pl.ds(s * 32, 32)],
                            cnts_hbm.at[pl.ds(b * (BUCKET // 16) + s * 32,
                                              32)])

            plsc.subcore_barrier()
            return 0
        lax.fori_loop(0, GB, bucket_body, 0)
        return 0
    lax.fori_loop(0, NG // NC, group_body, 0)


def _div_body(s_ref, c_ref, o_ref):
    o_ref[...] = s_ref[...] / jnp.maximum(c_ref[...], 1.0)


@jax.jit
def kernel(feat, grid_index):
    run = pl.kernel(
        _body,
        out_type=(
            jax.ShapeDtypeStruct((N_CELLS, C_DIM), jnp.float32),
            jax.ShapeDtypeStruct((N_CELLS // 16, 16), jnp.float32),
        ),
        mesh=plsc.VectorSubcoreMesh(core_axis_name="c", subcore_axis_name="s"),
        compiler_params=pltpu.CompilerParams(
            needs_layout_passes=False, use_tc_tiling_on_sc=False),
        scratch_types=[
            pltpu.VMEM((SB,), jnp.int32),
            pltpu.VMEM((GCAP,), jnp.int32),
            pltpu.VMEM((MCAP,), jnp.int32),
            pltpu.VMEM((BUCKET // 16, 16), jnp.float32),
            pltpu.VMEM(((BUCKET // 16) // 128, 128), jnp.int32),
            pltpu.VMEM((FB // 2,), jnp.int32),
            pltpu.VMEM((FB // 2,), jnp.int32),
            pltpu.VMEM((FB,), jnp.int32),
            pltpu.VMEM((FB, C_DIM), jnp.float32),
            pltpu.VMEM((FB // 2,), jnp.int32),
            pltpu.VMEM((FB // 2,), jnp.int32),
            pltpu.VMEM((FB,), jnp.int32),
            pltpu.VMEM((FB, C_DIM), jnp.float32),
            pltpu.VMEM((DC, C_DIM), jnp.float32),
            pltpu.VMEM((32, 16), jnp.float32),
            pltpu.VMEM_SHARED((BUCKET // 16, 16), jnp.float32),
            pltpu.VMEM_SHARED((BUCKET + 8, C_DIM), jnp.float32),
            pltpu.VMEM_SHARED((BUCKET // 16, 16), jnp.float32),
            pltpu.SemaphoreType.DMA,
            pltpu.SemaphoreType.DMA,
        ],
    )
    sums, cnts = run(feat, grid_index)
    cnts_col = cnts.reshape(N_CELLS, 1)
    RB = 2048
    return pl.pallas_call(
        _div_body,
        out_shape=jax.ShapeDtypeStruct((N_CELLS, C_DIM), jnp.float32),
        grid=(N_CELLS // RB,),
        in_specs=[
            pl.BlockSpec((RB, C_DIM), lambda i: (i, 0)),
            pl.BlockSpec((RB, 1), lambda i: (i, 0)),
        ],
        out_specs=pl.BlockSpec((RB, C_DIM), lambda i: (i, 0)),
    )(sums, cnts_col)


# speedup vs baseline: 1.0905x; 1.0229x over previous
import jax
import jax.numpy as jnp
from jax import lax
from jax.experimental import pallas as pl
from jax.experimental.pallas import tpu as pltpu
from jax.experimental.pallas import tpu_sc as plsc

N_PTS = 524288
C_DIM = 96
N_CELLS = 262144
NG = 8
G_SHIFT = 15
GB = 4
NB = NG * GB
BUCKET = N_CELLS // NB
L_SHIFT = 13
NC = 2
NS = 16
P = N_PTS // NS
GCAP = P + 16
FB = 128
MCAP = 2 * FB
TS = BUCKET // NS
DC = 64
SB = 2048


def _body(feat_hbm, gi_hbm, sums_hbm, cnts_hbm,
          gbuf, glist, mptr, lcnt, idbuf,
          gidxa0, gidxb0, lidx0, rowbuf0,
          gidxa1, gidxb1, lidx1, rowbuf1,
          dbuf, zc, zlbuf, acc, scnt, sem_g, sem_s, sem_d):
    c = lax.axis_index("c")
    s = lax.axis_index("s")
    tbase = s * P
    iota16 = lax.iota(jnp.int32, 16)
    zeros16 = jnp.zeros((16,), jnp.float32)
    ones16 = jnp.ones((16,), jnp.float32)
    neg16 = jnp.full((16,), -1, jnp.int32)

    def zdrow(r, _):
        for q in range(C_DIM // 16):
            dbuf[r, pl.ds(q * 16, 16)] = zeros16
        return 0
    lax.fori_loop(0, DC, zdrow, 0)

    def zcrow(r, _):
        zc[r] = zeros16
        return 0
    lax.fori_loop(0, 32, zcrow, 0)
    pltpu.sync_copy(zc, zlbuf.at[pl.ds(s * 32, 32)])
    plsc.subcore_barrier()

    def idrow(q, _):
        for i in range(8):
            idbuf[q, pl.ds(i * 16, 16)] = q * 128 + i * 16 + iota16
        return 0
    lax.fori_loop(0, (BUCKET // 16) // 128, idrow, 0)

    SLOTS = ((gidxa0, gidxb0, lidx0, rowbuf0),
             (gidxa1, gidxb1, lidx1, rowbuf1))

    def wait_scatter(rowb, lidxb):
        pltpu.make_async_copy(rowb, acc.at[lidxb], sem_s).wait()

    def issue_scatter(rowb, lidxb):
        pltpu.async_copy(rowb, acc.at[lidxb], sem_s, add=True)

    def wait_gathers(ga, gb, rowb):
        pltpu.make_async_copy(
            feat_hbm.at[ga], rowb.at[pl.ds(0, FB // 2)], sem_g).wait()
        pltpu.make_async_copy(
            feat_hbm.at[gb], rowb.at[pl.ds(FB // 2, FB // 2)], sem_g).wait()

    def issue_gathers(ga, gb, rowb):
        pltpu.async_copy(feat_hbm.at[ga], rowb.at[pl.ds(0, FB // 2)], sem_g)
        pltpu.async_copy(
            feat_hbm.at[gb], rowb.at[pl.ds(FB // 2, FB // 2)], sem_g)

    def flush_slot(slot, fcnt):
        ga, gb, lidxb, rowb = SLOTS[slot]
        oga, ogb, olidx, orow = SLOTS[1 - slot]

        @pl.when(fcnt >= 2)
        def _():
            wait_scatter(rowb, lidxb)
        for q in range(FB // 16):
            r16 = mptr[pl.ds(q * 16, 16)]
            valid = r16 >= 0
            loc = jnp.where(valid,
                            jnp.right_shift(r16, 15),
                            jnp.int32(BUCKET))
            rel = jnp.bitwise_and(r16, 32767)
            gi_half = ga if q < (FB // 32) else gb
            gi_half[pl.ds((q % (FB // 32)) * 16, 16)] = tbase + rel
            lidxb[pl.ds(q * 16, 16)] = loc

        @pl.when(fcnt >= 1)
        def _():
            wait_gathers(oga, ogb, orow)
            issue_scatter(orow, olidx)
        issue_gathers(ga, gb, rowb)

    def do_flush(fcnt):
        even = jnp.bitwise_and(fcnt, 1) == 0

        @pl.when(even)
        def _():
            flush_slot(0, fcnt)

        @pl.when(jnp.logical_not(even))
        def _():
            flush_slot(1, fcnt)

    def group_body(gi, _):
        g = c * (NG // NC) + gi

        def stream_issue(ch):
            p = jnp.bitwise_and(ch, 1)
            pltpu.async_copy(gi_hbm.at[pl.ds(tbase + ch * SB, SB)],
                             gbuf.at[pl.ds(p * SB, SB)], sem_d)

        def stream_wait(ch):
            p = jnp.bitwise_and(ch, 1)
            pltpu.make_async_copy(gi_hbm.at[pl.ds(tbase + ch * SB, SB)],
                                  gbuf.at[pl.ds(p * SB, SB)], sem_d).wait()

        stream_issue(jnp.int32(0))

        def stream_body(ch, gcnt):
            stream_wait(ch)

            @pl.when(ch + 1 < P // SB)
            def _():
                stream_issue(ch + 1)
            p = jnp.bitwise_and(ch, 1)

            def scan_body(i, cnt):
                v = gbuf[pl.ds(p * SB + i * 16, 16)]
                m = jnp.right_shift(v, G_SHIFT) == g
                rel = ch * SB + i * 16 + iota16
                e = jnp.bitwise_or(
                    jnp.left_shift(jnp.bitwise_and(v, 32767), 15), rel)
                pos = cnt + plsc.cumsum(m.astype(jnp.int32)) - 1
                plsc.store_scatter(glist, [pos], e, mask=m)
                return cnt + jnp.sum(m.astype(jnp.int32))
            return lax.fori_loop(0, SB // 16, scan_body, gcnt)
        gcnt = lax.fori_loop(0, P // SB, stream_body, jnp.int32(0))
        plsc.store_scatter(glist, [gcnt + iota16], neg16)
        gsteps = jnp.right_shift(gcnt + 15, 4)

        def bucket_body(sub, _):
            b = g * GB + sub

            pltpu.sync_copy(zlbuf, lcnt)
            for kk in range(TS // DC):
                pltpu.sync_copy(dbuf, acc.at[pl.ds(s * TS + kk * DC, DC)])
            pltpu.sync_copy(lcnt.at[pl.ds(0, 32)],
                            scnt.at[pl.ds(s * 32, 32)])
            plsc.subcore_barrier()

            def bscan(i, carry):
                cnt, fcnt = carry
                e = glist[pl.ds(i * 16, 16)]
                cell15 = jnp.right_shift(e, 15)
                m = jnp.logical_and(
                    jnp.right_shift(cell15, L_SHIFT) == sub, e >= 0)
                loc = jnp.bitwise_and(cell15, BUCKET - 1)
                plsc.addupdate_scatter(
                    lcnt, [jnp.right_shift(loc, 4),
                           jnp.bitwise_and(loc, 15)], ones16, mask=m)
                packed = jnp.bitwise_or(
                    jnp.left_shift(loc, 15), jnp.bitwise_and(e, 32767))
                pos = cnt + plsc.cumsum(m.astype(jnp.int32)) - 1
                plsc.store_scatter(mptr, [pos], packed, mask=m)
                cnt2 = cnt + jnp.sum(m.astype(jnp.int32))
                full = cnt2 >= FB

                @pl.when(full)
                def _():
                    do_flush(fcnt)
                    ov = mptr[pl.ds(FB, 16)]
                    mptr[pl.ds(0, 16)] = ov
                cnt3 = jnp.where(full, cnt2 - FB, cnt2)
                return cnt3, fcnt + full.astype(jnp.int32)
            cnt, fcnt = lax.fori_loop(0, gsteps, bscan,
                                      (jnp.int32(0), jnp.int32(0)))

            def pad_body(q, _):
                plsc.store_scatter(mptr, [cnt + q * 16 + iota16], neg16)
                return 0
            lax.fori_loop(0, 8, pad_body, 0)

            @pl.when(cnt > 0)
            def _():
                do_flush(fcnt)
            ftot = fcnt + (cnt > 0).astype(jnp.int32)

            def drain_par(slot, nf):
                ga, gb, lidxb, rowb = SLOTS[slot]
                oga, ogb, olidx, orow = SLOTS[1 - slot]
                wait_gathers(ga, gb, rowb)
                issue_scatter(rowb, lidxb)

                @pl.when(nf >= 2)
                def _():
                    wait_scatter(orow, olidx)
                wait_scatter(rowb, lidxb)

            lastp = jnp.bitwise_and(ftot - 1, 1)

            @pl.when(jnp.logical_and(ftot >= 1, lastp == 0))
            def _():
                drain_par(0, ftot)

            @pl.when(jnp.logical_and(ftot >= 1, lastp == 1))
            def _():
                drain_par(1, ftot)

            for q in range((BUCKET // 16) // 128):
                pltpu.sync_copy(lcnt.at[pl.ds(q * 128, 128)],
                                scnt.at[idbuf.at[q]], add=True)
            plsc.subcore_barrier()

            def dchunk(kk, _):
                row0 = s * TS + kk * DC
                pltpu.sync_copy(acc.at[pl.ds(row0, DC)],
                                sums_hbm.at[pl.ds(b * BUCKET + row0, DC)])
                return 0
            lax.fori_loop(0, TS // DC, dchunk, 0)
            pltpu.sync_copy(scnt.at[pl.ds(s * 32, 32)],
                            cnts_hbm.at[pl.ds(b * (BUCKET // 16) + s * 32,
                                              32)])

            plsc.subcore_barrier()
            return 0
        lax.fori_loop(0, GB, bucket_body, 0)
        return 0
    lax.fori_loop(0, NG // NC, group_body, 0)


def _div_body(s_ref, c_ref, o_ref):
    o_ref[...] = s_ref[...] / jnp.maximum(c_ref[...], 1.0)


@jax.jit
def kernel(feat, grid_index):
    run = pl.kernel(
        _body,
        out_type=(
            jax.ShapeDtypeStruct((N_CELLS, C_DIM), jnp.float32),
            jax.ShapeDtypeStruct((N_CELLS // 16, 16), jnp.float32),
        ),
        mesh=plsc.VectorSubcoreMesh(core_axis_name="c", subcore_axis_name="s"),
        compiler_params=pltpu.CompilerParams(
            needs_layout_passes=False, use_tc_tiling_on_sc=False),
        scratch_types=[
            pltpu.VMEM((2 * SB,), jnp.int32),
            pltpu.VMEM((GCAP,), jnp.int32),
            pltpu.VMEM((MCAP,), jnp.int32),
            pltpu.VMEM((BUCKET // 16, 16), jnp.float32),
            pltpu.VMEM(((BUCKET // 16) // 128, 128), jnp.int32),
            pltpu.VMEM((FB // 2,), jnp.int32),
            pltpu.VMEM((FB // 2,), jnp.int32),
            pltpu.VMEM((FB,), jnp.int32),
            pltpu.VMEM((FB, C_DIM), jnp.float32),
            pltpu.VMEM((FB // 2,), jnp.int32),
            pltpu.VMEM((FB // 2,), jnp.int32),
            pltpu.VMEM((FB,), jnp.int32),
            pltpu.VMEM((FB, C_DIM), jnp.float32),
            pltpu.VMEM((DC, C_DIM), jnp.float32),
            pltpu.VMEM((32, 16), jnp.float32),
            pltpu.VMEM_SHARED((BUCKET // 16, 16), jnp.float32),
            pltpu.VMEM_SHARED((BUCKET + 8, C_DIM), jnp.float32),
            pltpu.VMEM_SHARED((BUCKET // 16, 16), jnp.float32),
            pltpu.SemaphoreType.DMA,
            pltpu.SemaphoreType.DMA,
            pltpu.SemaphoreType.DMA,
        ],
    )
    sums, cnts = run(feat, grid_index)
    cnts_col = cnts.reshape(N_CELLS, 1)
    RB = 2048
    return pl.pallas_call(
        _div_body,
        out_shape=jax.ShapeDtypeStruct((N_CELLS, C_DIM), jnp.float32),
        grid=(N_CELLS // RB,),
        in_specs=[
            pl.BlockSpec((RB, C_DIM), lambda i: (i, 0)),
            pl.BlockSpec((RB, 1), lambda i: (i, 0)),
        ],
        out_specs=pl.BlockSpec((RB, C_DIM), lambda i: (i, 0)),
    )(sums, cnts_col)


# speedup vs baseline: 1.0911x; 1.0006x over previous
import jax
import jax.numpy as jnp
from jax import lax
from jax.experimental import pallas as pl
from jax.experimental.pallas import tpu as pltpu
from jax.experimental.pallas import tpu_sc as plsc

N_PTS = 524288
C_DIM = 96
N_CELLS = 262144
NG = 8
G_SHIFT = 15
GB = 4
NB = NG * GB
BUCKET = N_CELLS // NB
L_SHIFT = 13
NC = 2
NS = 16
P = N_PTS // NS
GCAP = P + 16
FB = 128
MCAP = 2 * FB
TS = BUCKET // NS
DC = 64
SB = 2048


def _body(feat_hbm, gi_hbm, sums_hbm, cnts_hbm,
          gbuf, glist, mptr, lcnt, idbuf,
          gidxa0, gidxb0, lidx0, rowbuf0,
          gidxa1, gidxb1, lidx1, rowbuf1,
          dbuf, zc, zlbuf, acc, scnt, sem_g, sem_s, sem_d):
    c = lax.axis_index("c")
    s = lax.axis_index("s")
    tbase = s * P
    iota16 = lax.iota(jnp.int32, 16)
    zeros16 = jnp.zeros((16,), jnp.float32)
    ones16 = jnp.ones((16,), jnp.float32)
    neg16 = jnp.full((16,), -1, jnp.int32)

    def zdrow(r, _):
        for q in range(C_DIM // 16):
            dbuf[r, pl.ds(q * 16, 16)] = zeros16
        return 0
    lax.fori_loop(0, DC, zdrow, 0)

    def zcrow(r, _):
        zc[r] = zeros16
        return 0
    lax.fori_loop(0, 32, zcrow, 0)
    pltpu.sync_copy(zc, zlbuf.at[pl.ds(s * 32, 32)])
    plsc.subcore_barrier()

    def idrow(q, _):
        for i in range(8):
            idbuf[q, pl.ds(i * 16, 16)] = q * 128 + i * 16 + iota16
        return 0
    lax.fori_loop(0, (BUCKET // 16) // 128, idrow, 0)

    SLOTS = ((gidxa0, gidxb0, lidx0, rowbuf0),
             (gidxa1, gidxb1, lidx1, rowbuf1))

    def wait_scatter(rowb, lidxb):
        pltpu.make_async_copy(rowb, acc.at[lidxb], sem_s).wait()

    def issue_scatter(rowb, lidxb):
        pltpu.async_copy(rowb, acc.at[lidxb], sem_s, add=True)

    def wait_gathers(ga, gb, rowb):
        pltpu.make_async_copy(
            feat_hbm.at[ga], rowb.at[pl.ds(0, FB // 2)], sem_g).wait()
        pltpu.make_async_copy(
            feat_hbm.at[gb], rowb.at[pl.ds(FB // 2, FB // 2)], sem_g).wait()

    def issue_gathers(ga, gb, rowb):
        pltpu.async_copy(feat_hbm.at[ga], rowb.at[pl.ds(0, FB // 2)], sem_g)
        pltpu.async_copy(
            feat_hbm.at[gb], rowb.at[pl.ds(FB // 2, FB // 2)], sem_g)

    def flush_slot(slot, fcnt):
        ga, gb, lidxb, rowb = SLOTS[slot]
        oga, ogb, olidx, orow = SLOTS[1 - slot]

        @pl.when(fcnt >= 2)
        def _():
            wait_scatter(rowb, lidxb)
        for q in range(FB // 16):
            r16 = mptr[pl.ds(q * 16, 16)]
            valid = r16 >= 0
            loc = jnp.where(valid,
                            jnp.right_shift(r16, 15),
                            jnp.int32(BUCKET))
            rel = jnp.bitwise_and(r16, 32767)
            gi_half = ga if q < (FB // 32) else gb
            gi_half[pl.ds((q % (FB // 32)) * 16, 16)] = tbase + rel
            lidxb[pl.ds(q * 16, 16)] = loc

        @pl.when(fcnt >= 1)
        def _():
            wait_gathers(oga, ogb, orow)
            issue_scatter(orow, olidx)
        issue_gathers(ga, gb, rowb)

    def do_flush(fcnt):
        even = jnp.bitwise_and(fcnt, 1) == 0

        @pl.when(even)
        def _():
            flush_slot(0, fcnt)

        @pl.when(jnp.logical_not(even))
        def _():
            flush_slot(1, fcnt)

    def group_body(gi, _):
        g = c * (NG // NC) + gi

        def stream_issue(ch):
            p = jnp.bitwise_and(ch, 1)
            pltpu.async_copy(gi_hbm.at[pl.ds(tbase + ch * SB, SB)],
                             gbuf.at[pl.ds(p * SB, SB)], sem_d)

        def stream_wait(ch):
            p = jnp.bitwise_and(ch, 1)
            pltpu.make_async_copy(gi_hbm.at[pl.ds(tbase + ch * SB, SB)],
                                  gbuf.at[pl.ds(p * SB, SB)], sem_d).wait()

        stream_issue(jnp.int32(0))

        def stream_body(ch, gcnt):
            stream_wait(ch)

            @pl.when(ch + 1 < P // SB)
            def _():
                stream_issue(ch + 1)
            p = jnp.bitwise_and(ch, 1)

            def scan_body(i, cnt):
                v = gbuf[pl.ds(p * SB + i * 16, 16)]
                m = jnp.right_shift(v, G_SHIFT) == g
                rel = ch * SB + i * 16 + iota16
                e = jnp.bitwise_or(
                    jnp.left_shift(jnp.bitwise_and(v, 32767), 15), rel)
                pos = cnt + plsc.cumsum(m.astype(jnp.int32)) - 1
                plsc.store_scatter(glist, [pos], e, mask=m)
                return cnt + jnp.sum(m.astype(jnp.int32))
            return lax.fori_loop(0, SB // 16, scan_body, gcnt)
        gcnt = lax.fori_loop(0, P // SB, stream_body, jnp.int32(0))
        plsc.store_scatter(glist, [gcnt + iota16], neg16)
        gsteps = jnp.right_shift(gcnt + 15, 4)

        def bucket_body(sub, _):
            b = g * GB + sub

            pltpu.sync_copy(zlbuf, lcnt)
            for kk in range(TS // DC):
                pltpu.sync_copy(dbuf, acc.at[pl.ds(s * TS + kk * DC, DC)])
            pltpu.sync_copy(lcnt.at[pl.ds(0, 32)],
                            scnt.at[pl.ds(s * 32, 32)])
            plsc.subcore_barrier()

            def bscan(i, carry):
                cnt, fcnt = carry
                e = glist[pl.ds(i * 16, 16)]
                cell15 = jnp.right_shift(e, 15)
                m = jnp.logical_and(
                    jnp.right_shift(cell15, L_SHIFT) == sub, e >= 0)
                loc = jnp.bitwise_and(cell15, BUCKET - 1)
                plsc.addupdate_scatter(
                    lcnt, [jnp.right_shift(loc, 4),
                           jnp.bitwise_and(loc, 15)], ones16, mask=m)
                packed = jnp.bitwise_or(
                    jnp.left_shift(loc, 15), jnp.bitwise_and(e, 32767))
                pos = cnt + plsc.cumsum(m.astype(jnp.int32)) - 1
                plsc.store_scatter(mptr, [pos], packed, mask=m)
                cnt2 = cnt + jnp.sum(m.astype(jnp.int32))
                full = cnt2 >= FB

                @pl.when(full)
                def _():
                    do_flush(fcnt)
                    ov = mptr[pl.ds(FB, 16)]
                    mptr[pl.ds(0, 16)] = ov
                cnt3 = jnp.where(full, cnt2 - FB, cnt2)
                return cnt3, fcnt + full.astype(jnp.int32)
            cnt, fcnt = lax.fori_loop(0, gsteps, bscan,
                                      (jnp.int32(0), jnp.int32(0)))

            def pad_body(q, _):
                plsc.store_scatter(mptr, [cnt + q * 16 + iota16], neg16)
                return 0
            lax.fori_loop(0, 8, pad_body, 0)

            @pl.when(cnt > 0)
            def _():
                do_flush(fcnt)
            ftot = fcnt + (cnt > 0).astype(jnp.int32)

            def drain_par(slot, nf):
                ga, gb, lidxb, rowb = SLOTS[slot]
                oga, ogb, olidx, orow = SLOTS[1 - slot]
                wait_gathers(ga, gb, rowb)
                issue_scatter(rowb, lidxb)

                @pl.when(nf >= 2)
                def _():
                    wait_scatter(orow, olidx)
                wait_scatter(rowb, lidxb)

            lastp = jnp.bitwise_and(ftot - 1, 1)

            @pl.when(jnp.logical_and(ftot >= 1, lastp == 0))
            def _():
                drain_par(0, ftot)

            @pl.when(jnp.logical_and(ftot >= 1, lastp == 1))
            def _():
                drain_par(1, ftot)

            for q in range((BUCKET // 16) // 128):
                pltpu.sync_copy(lcnt.at[pl.ds(q * 128, 128)],
                                scnt.at[idbuf.at[q]], add=True)
            plsc.subcore_barrier()

            def dchunk(kk, _):
                row0 = s * TS + kk * DC
                pltpu.sync_copy(acc.at[pl.ds(row0, DC)],
                                sums_hbm.at[pl.ds(b * BUCKET + row0, DC)])
                return 0
            lax.fori_loop(0, TS // DC, dchunk, 0)
            pltpu.sync_copy(scnt.at[pl.ds(s * 32, 32)],
                            cnts_hbm.at[pl.ds(b * (BUCKET // 16) + s * 32,
                                              32)])
            return 0
        lax.fori_loop(0, GB, bucket_body, 0)
        return 0
    lax.fori_loop(0, NG // NC, group_body, 0)


def _div_body(s_ref, c_ref, o_ref):
    o_ref[...] = s_ref[...] / jnp.maximum(c_ref[...], 1.0)


@jax.jit
def kernel(feat, grid_index):
    run = pl.kernel(
        _body,
        out_type=(
            jax.ShapeDtypeStruct((N_CELLS, C_DIM), jnp.float32),
            jax.ShapeDtypeStruct((N_CELLS // 16, 16), jnp.float32),
        ),
        mesh=plsc.VectorSubcoreMesh(core_axis_name="c", subcore_axis_name="s"),
        compiler_params=pltpu.CompilerParams(
            needs_layout_passes=False, use_tc_tiling_on_sc=False),
        scratch_types=[
            pltpu.VMEM((2 * SB,), jnp.int32),
            pltpu.VMEM((GCAP,), jnp.int32),
            pltpu.VMEM((MCAP,), jnp.int32),
            pltpu.VMEM((BUCKET // 16, 16), jnp.float32),
            pltpu.VMEM(((BUCKET // 16) // 128, 128), jnp.int32),
            pltpu.VMEM((FB // 2,), jnp.int32),
            pltpu.VMEM((FB // 2,), jnp.int32),
            pltpu.VMEM((FB,), jnp.int32),
            pltpu.VMEM((FB, C_DIM), jnp.float32),
            pltpu.VMEM((FB // 2,), jnp.int32),
            pltpu.VMEM((FB // 2,), jnp.int32),
            pltpu.VMEM((FB,), jnp.int32),
            pltpu.VMEM((FB, C_DIM), jnp.float32),
            pltpu.VMEM((DC, C_DIM), jnp.float32),
            pltpu.VMEM((32, 16), jnp.float32),
            pltpu.VMEM_SHARED((BUCKET // 16, 16), jnp.float32),
            pltpu.VMEM_SHARED((BUCKET + 8, C_DIM), jnp.float32),
            pltpu.VMEM_SHARED((BUCKET // 16, 16), jnp.float32),
            pltpu.SemaphoreType.DMA,
            pltpu.SemaphoreType.DMA,
            pltpu.SemaphoreType.DMA,
        ],
    )
    sums, cnts = run(feat, grid_index)
    cnts_col = cnts.reshape(N_CELLS, 1)
    RB = 2048
    return pl.pallas_call(
        _div_body,
        out_shape=jax.ShapeDtypeStruct((N_CELLS, C_DIM), jnp.float32),
        grid=(N_CELLS // RB,),
        in_specs=[
            pl.BlockSpec((RB, C_DIM), lambda i: (i, 0)),
            pl.BlockSpec((RB, 1), lambda i: (i, 0)),
        ],
        out_specs=pl.BlockSpec((RB, C_DIM), lambda i: (i, 0)),
    )(sums, cnts_col)
